# baseline (speedup 1.0000x reference)
import functools

import jax
import jax.numpy as jnp
from jax import lax
from jax.experimental import pallas as pl
from jax.experimental.pallas import tpu as pltpu
from jax.experimental.pallas import tpu_sc as plsc

F = 16
L = 16
NC, NS = 2, 16
NW = NC * NS
B = 16384
BPW = B // NW
CHUNK = 128
NCHUNK = BPW // CHUNK


def _sc_embed_dot(user2d, item2d, user_table, item_table):
    mesh = plsc.VectorSubcoreMesh(core_axis_name="c", subcore_axis_name="s")

    @functools.partial(
        pl.kernel,
        out_type=jax.ShapeDtypeStruct((B,), jnp.float32),
        mesh=mesh,
        scratch_types=[
            pltpu.VMEM((NCHUNK, CHUNK), jnp.int32),
            pltpu.VMEM((NCHUNK, CHUNK), jnp.int32),
            pltpu.VMEM((BPW, F), jnp.float32),
            pltpu.VMEM((BPW, F), jnp.float32),
            pltpu.VMEM((BPW,), jnp.float32),
            pltpu.SemaphoreType.DMA,
        ],
        compiler_params=pltpu.CompilerParams(
            needs_layout_passes=False, use_tc_tiling_on_sc=False),
    )
    def k(user_hbm, item_hbm, ut_hbm, it_hbm, out_hbm,
          uidx, iidx, urows, irows, outv, sem):
        wid = lax.axis_index("s") * NC + lax.axis_index("c")
        pltpu.sync_copy(user_hbm.at[pl.ds(wid * NCHUNK, NCHUNK)], uidx)
        pltpu.sync_copy(item_hbm.at[pl.ds(wid * NCHUNK, NCHUNK)], iidx)
        copies = []
        for j in range(NCHUNK):
            copies.append(pltpu.async_copy(
                ut_hbm.at[uidx.at[j]], urows.at[pl.ds(j * CHUNK, CHUNK)], sem))
            copies.append(pltpu.async_copy(
                it_hbm.at[iidx.at[j]], irows.at[pl.ds(j * CHUNK, CHUNK)], sem))
        for c in copies:
            c.wait()

        def blk_body(blk, carry):
            row = blk * L + lax.iota(jnp.int32, L)
            acc = jnp.zeros((L,), jnp.float32)
            for f in range(F):
                col = jnp.full((L,), f, dtype=jnp.int32)
                acc = acc + (plsc.load_gather(urows, [row, col]) *
                             plsc.load_gather(irows, [row, col]))
            outv[pl.ds(blk * L, L)] = acc
            return carry

        lax.fori_loop(0, BPW // L, blk_body, 0)
        pltpu.sync_copy(outv, out_hbm.at[pl.ds(wid * BPW, BPW)])

    return k(user2d, item2d, user_table, item_table)


def kernel(user, item, user_table, item_table):
    user2d = user.reshape(NW * NCHUNK, CHUNK)
    item2d = item.reshape(NW * NCHUNK, CHUNK)
    out = _sc_embed_dot(user2d, item2d, user_table, item_table)
    return out[:, None]
